# Initial kernel scaffold; baseline (speedup 1.0000x reference)
import jax
import jax.numpy as jnp
from jax import lax
from jax.experimental import pallas as pl
from jax.experimental.pallas import tpu as pltpu
from jax.experimental.pallas import tpu_sc as plsc

N = 10000
D = 128
E = 320000
C = 64

NC = 2
NS = 16
NW = NC * NS
EPW = E // NW
CHUNK = 80
NCHUNK = EPW // CHUNK
ROWS_PT = N // NS
DEG_PAD = 10240
DROWS_PT = DEG_PAD // NS


def _deg_body(dst_hbm, ones_hbm, z_hbm, out_hbm, didx, ones_v, dacc):
    c = lax.axis_index("c")
    s = lax.axis_index("s")
    wid = c * NS + s
    r0 = s * DROWS_PT
    pltpu.sync_copy(z_hbm.at[pl.ds(r0, DROWS_PT)], dacc.at[pl.ds(r0, DROWS_PT)])
    pltpu.sync_copy(ones_hbm, ones_v)
    pltpu.sync_copy(dst_hbm.at[wid], didx)
    plsc.subcore_barrier()

    def body(j, carry):
        pltpu.sync_copy(ones_v, dacc.at[didx.at[j]], add=True)
        return carry

    lax.fori_loop(0, NCHUNK, body, 0)
    plsc.subcore_barrier()
    pltpu.sync_copy(dacc.at[pl.ds(r0, DROWS_PT)],
                    out_hbm.at[c, pl.ds(r0, DROWS_PT)])


def _prop_body(g_hbm, src_hbm, dst_hbm, z_hbm, out_hbm,
               sidx, didx, buf0, buf1, sem0, sem1, acc):
    c = lax.axis_index("c")
    s = lax.axis_index("s")
    wid = c * NS + s
    r0 = s * ROWS_PT
    pltpu.sync_copy(z_hbm.at[pl.ds(r0, ROWS_PT)], acc.at[pl.ds(r0, ROWS_PT)])
    pltpu.sync_copy(src_hbm.at[wid], sidx)
    pltpu.sync_copy(dst_hbm.at[wid], didx)
    pltpu.async_copy(g_hbm.at[sidx.at[0]], buf0, sem0)
    plsc.subcore_barrier()

    bufs = (buf0, buf1)
    sems = (sem0, sem1)

    def body(i, carry):
        g2 = 2 * i
        for b in range(2):
            j = g2 + b
            buf, sem = bufs[b], sems[b]
            nbuf, nsem = bufs[1 - b], sems[1 - b]
            pltpu.make_async_copy(g_hbm.at[sidx.at[j]], buf, sem).wait()
            pltpu.async_copy(g_hbm.at[sidx.at[j + 1]], nbuf, nsem)
            pltpu.sync_copy(buf, acc.at[didx.at[j]], add=True)
        return carry

    lax.fori_loop(0, (NCHUNK - 1) // 2, body, 0)
    pltpu.make_async_copy(g_hbm.at[sidx.at[NCHUNK - 1]], buf0, sem0).wait()
    pltpu.sync_copy(buf0, acc.at[didx.at[NCHUNK - 1]], add=True)
    plsc.subcore_barrier()
    pltpu.sync_copy(acc.at[pl.ds(r0, ROWS_PT)],
                    out_hbm.at[c, pl.ds(r0, ROWS_PT)])


RB = 1250


def _mm_body(x_ref, w_ref, d0_ref, d1_ref, g0_ref, da_ref, db_ref):
    deg = d0_ref[:, 0:1] + d1_ref[:, 0:1] + 1.0
    dis = lax.rsqrt(deg)
    y = lax.dot_general(x_ref[...], w_ref[...], (((1,), (1,)), ((), ())),
                        preferred_element_type=jnp.float32,
                        precision=lax.Precision.HIGHEST)
    g0_ref[...] = y * dis
    da_ref[...] = 1.0 / deg
    db_ref[...] = dis


def _comb_body(p_ref, g_ref, sc_ref, o_ref):
    p = p_ref[...]
    o_ref[...] = (p[0] + p[1] + g_ref[...]) * sc_ref[...]


def _out_body(p_ref, g_ref, sc_ref, b_ref, o_ref):
    p = p_ref[...]
    o_ref[...] = (p[0] + p[1] + g_ref[...]) * sc_ref[...] + b_ref[...]


def _prop(g, src, dst, z64, mesh):
    return pl.kernel(
        _prop_body,
        out_type=jax.ShapeDtypeStruct((NC, N, C), jnp.float32),
        mesh=mesh,
        scratch_types=[
            pltpu.VMEM((NCHUNK, CHUNK), jnp.int32),
            pltpu.VMEM((NCHUNK, CHUNK), jnp.int32),
            pltpu.VMEM((CHUNK, C), jnp.float32),
            pltpu.VMEM((CHUNK, C), jnp.float32),
            pltpu.SemaphoreType.DMA,
            pltpu.SemaphoreType.DMA,
            pltpu.VMEM_SHARED((N, C), jnp.float32),
        ],
    )(g, src, dst, z64)


def _combine(partial, g, scale, bias=None):
    if bias is None:
        body, extra = _comb_body, []
        in_specs_extra = []
    else:
        body, extra = _out_body, [bias]
        in_specs_extra = [pl.BlockSpec((1, C), lambda i: (0, 0))]
    return pl.pallas_call(
        body,
        grid=(N // RB,),
        in_specs=[
            pl.BlockSpec((NC, RB, C), lambda i: (0, i, 0)),
            pl.BlockSpec((RB, C), lambda i: (i, 0)),
            pl.BlockSpec((RB, 1), lambda i: (i, 0)),
        ] + in_specs_extra,
        out_specs=pl.BlockSpec((RB, C), lambda i: (i, 0)),
        out_shape=jax.ShapeDtypeStruct((N, C), jnp.float32),
    )(partial, g, scale, *extra)


def kernel(x, edge_index, W, b):
    ei = edge_index.astype(jnp.int32)
    src = ei[0].reshape(NW, NCHUNK, CHUNK)
    dst = ei[1].reshape(NW, NCHUNK, CHUNK)
    z64 = jnp.zeros((N, C), jnp.float32)
    z16 = jnp.zeros((DEG_PAD, 16), jnp.float32)
    ones16 = jnp.ones((CHUNK, 16), jnp.float32)
    mesh = plsc.VectorSubcoreMesh(core_axis_name="c", subcore_axis_name="s",
                                  num_cores=NC, num_subcores=NS)

    deg_partial = pl.kernel(
        _deg_body,
        out_type=jax.ShapeDtypeStruct((NC, DEG_PAD, 16), jnp.float32),
        mesh=mesh,
        scratch_types=[
            pltpu.VMEM((NCHUNK, CHUNK), jnp.int32),
            pltpu.VMEM((CHUNK, 16), jnp.float32),
            pltpu.VMEM_SHARED((DEG_PAD, 16), jnp.float32),
        ],
    )(dst, ones16, z16)

    g0, d_a, d_b = pl.pallas_call(
        _mm_body,
        grid=(N // RB,),
        in_specs=[
            pl.BlockSpec((RB, D), lambda i: (i, 0)),
            pl.BlockSpec((C, D), lambda i: (0, 0)),
            pl.BlockSpec((RB, 16), lambda i: (i, 0)),
            pl.BlockSpec((RB, 16), lambda i: (i, 0)),
        ],
        out_specs=[
            pl.BlockSpec((RB, C), lambda i: (i, 0)),
            pl.BlockSpec((RB, 1), lambda i: (i, 0)),
            pl.BlockSpec((RB, 1), lambda i: (i, 0)),
        ],
        out_shape=[
            jax.ShapeDtypeStruct((N, C), jnp.float32),
            jax.ShapeDtypeStruct((N, 1), jnp.float32),
            jax.ShapeDtypeStruct((N, 1), jnp.float32),
        ],
    )(x, W, deg_partial[0], deg_partial[1])

    p1 = _prop(g0, src, dst, z64, mesh)
    g1 = _combine(p1, g0, d_a)
    p2 = _prop(g1, src, dst, z64, mesh)
    return _combine(p2, g1, d_b, bias=b.reshape(1, C))


# speedup vs baseline: 26.8721x; 26.8721x over previous
import jax
import jax.numpy as jnp
from jax import lax
from jax.experimental import pallas as pl
from jax.experimental.pallas import tpu as pltpu
from jax.experimental.pallas import tpu_sc as plsc

N = 10000
D = 128
E = 320000
C = 64

NC = 2
NS = 16
NW = NC * NS
EPW = E // NW
CHUNK = 80
NCHUNK = EPW // CHUNK
PN = 10240
ROWS_PT = PN // NS
DEG_PAD = 10240
DROWS_PT = DEG_PAD // NS


def _deg_body(dst_hbm, ones_hbm, z_hbm, out_hbm, didx, ones_v, dacc):
    c = lax.axis_index("c")
    s = lax.axis_index("s")
    wid = c * NS + s
    r0 = s * DROWS_PT
    pltpu.sync_copy(z_hbm.at[pl.ds(r0, DROWS_PT)], dacc.at[pl.ds(r0, DROWS_PT)])
    pltpu.sync_copy(ones_hbm, ones_v)
    pltpu.sync_copy(dst_hbm.at[wid], didx)
    plsc.subcore_barrier()

    def body(j, carry):
        pltpu.sync_copy(ones_v, dacc.at[didx.at[j]], add=True)
        return carry

    lax.fori_loop(0, NCHUNK, body, 0)
    plsc.subcore_barrier()
    pltpu.sync_copy(dacc.at[pl.ds(r0, DROWS_PT)],
                    out_hbm.at[c, pl.ds(r0, DROWS_PT)])


def _prop_body(g_hbm, src_hbm, dst_hbm, z_hbm, out_hbm,
               sidx, didx, buf0, buf1, sem0, sem1, acc):
    c = lax.axis_index("c")
    s = lax.axis_index("s")
    wid = c * NS + s
    r0 = s * ROWS_PT
    pltpu.sync_copy(z_hbm.at[pl.ds(r0, ROWS_PT)], acc.at[pl.ds(r0, ROWS_PT)])
    pltpu.sync_copy(src_hbm.at[wid], sidx)
    pltpu.sync_copy(dst_hbm.at[wid], didx)
    pltpu.async_copy(g_hbm.at[sidx.at[0]], buf0, sem0)
    plsc.subcore_barrier()

    bufs = (buf0, buf1)
    sems = (sem0, sem1)

    def body(i, carry):
        g2 = 2 * i
        for b in range(2):
            j = g2 + b
            buf, sem = bufs[b], sems[b]
            nbuf, nsem = bufs[1 - b], sems[1 - b]
            pltpu.make_async_copy(g_hbm.at[sidx.at[j]], buf, sem).wait()
            pltpu.async_copy(g_hbm.at[sidx.at[j + 1]], nbuf, nsem)
            pltpu.sync_copy(buf, acc.at[didx.at[j]], add=True)
        return carry

    lax.fori_loop(0, (NCHUNK - 1) // 2, body, 0)
    pltpu.make_async_copy(g_hbm.at[sidx.at[NCHUNK - 1]], buf0, sem0).wait()
    pltpu.sync_copy(buf0, acc.at[didx.at[NCHUNK - 1]], add=True)
    plsc.subcore_barrier()
    pltpu.sync_copy(acc.at[pl.ds(r0, ROWS_PT)],
                    out_hbm.at[c, pl.ds(r0, ROWS_PT)])


RB = 1000


def _mm_body(x_ref, w_ref, d0_ref, d1_ref, g0_ref, da_ref, db_ref):
    deg = d0_ref[:, 0:1] + d1_ref[:, 0:1] + 1.0
    dis = lax.rsqrt(deg)
    y = lax.dot_general(x_ref[...], w_ref[...], (((1,), (1,)), ((), ())),
                        preferred_element_type=jnp.float32,
                        precision=lax.Precision.HIGHEST)
    g0_ref[...] = y * dis
    da_ref[...] = 1.0 / deg
    db_ref[...] = dis


def _comb_body(p_ref, g_ref, sc_ref, o_ref):
    p = p_ref[...]
    o_ref[...] = (p[0] + p[1] + g_ref[...]) * sc_ref[...]


def _out_body(p_ref, g_ref, sc_ref, b_ref, o_ref):
    p = p_ref[...]
    o_ref[...] = (p[0] + p[1] + g_ref[...]) * sc_ref[...] + b_ref[...]


def _prop(g, src, dst, z64, mesh):
    return pl.kernel(
        _prop_body,
        out_type=jax.ShapeDtypeStruct((NC, PN, C), jnp.float32),
        mesh=mesh,
        scratch_types=[
            pltpu.VMEM((NCHUNK, CHUNK), jnp.int32),
            pltpu.VMEM((NCHUNK, CHUNK), jnp.int32),
            pltpu.VMEM((CHUNK, C), jnp.float32),
            pltpu.VMEM((CHUNK, C), jnp.float32),
            pltpu.SemaphoreType.DMA,
            pltpu.SemaphoreType.DMA,
            pltpu.VMEM_SHARED((PN, C), jnp.float32),
        ],
        compiler_params=pltpu.CompilerParams(use_tc_tiling_on_sc=False),
    )(g, src, dst, z64)


def _combine(partial, g, scale, bias=None):
    if bias is None:
        body, extra = _comb_body, []
        in_specs_extra = []
    else:
        body, extra = _out_body, [bias]
        in_specs_extra = [pl.BlockSpec((1, C), lambda i: (0, 0))]
    return pl.pallas_call(
        body,
        grid=(N // RB,),
        in_specs=[
            pl.BlockSpec((NC, RB, C), lambda i: (0, i, 0)),
            pl.BlockSpec((RB, C), lambda i: (i, 0)),
            pl.BlockSpec((RB, 1), lambda i: (i, 0)),
        ] + in_specs_extra,
        out_specs=pl.BlockSpec((RB, C), lambda i: (i, 0)),
        out_shape=jax.ShapeDtypeStruct((N, C), jnp.float32),
    )(partial, g, scale, *extra)


def kernel(x, edge_index, W, b):
    ei = edge_index.astype(jnp.int32)
    src = ei[0].reshape(NW, NCHUNK, CHUNK)
    dst = ei[1].reshape(NW, NCHUNK, CHUNK)
    z64 = jnp.zeros((PN, C), jnp.float32)
    z16 = jnp.zeros((DEG_PAD, 16), jnp.float32)
    ones16 = jnp.ones((CHUNK, 16), jnp.float32)
    mesh = plsc.VectorSubcoreMesh(core_axis_name="c", subcore_axis_name="s",
                                  num_cores=NC, num_subcores=NS)

    deg_partial = pl.kernel(
        _deg_body,
        out_type=jax.ShapeDtypeStruct((NC, DEG_PAD, 16), jnp.float32),
        mesh=mesh,
        scratch_types=[
            pltpu.VMEM((NCHUNK, CHUNK), jnp.int32),
            pltpu.VMEM((CHUNK, 16), jnp.float32),
            pltpu.VMEM_SHARED((DEG_PAD, 16), jnp.float32),
        ],
        compiler_params=pltpu.CompilerParams(use_tc_tiling_on_sc=False),
    )(dst, ones16, z16)

    g0, d_a, d_b = pl.pallas_call(
        _mm_body,
        grid=(N // RB,),
        in_specs=[
            pl.BlockSpec((RB, D), lambda i: (i, 0)),
            pl.BlockSpec((C, D), lambda i: (0, 0)),
            pl.BlockSpec((RB, 16), lambda i: (i, 0)),
            pl.BlockSpec((RB, 16), lambda i: (i, 0)),
        ],
        out_specs=[
            pl.BlockSpec((RB, C), lambda i: (i, 0)),
            pl.BlockSpec((RB, 1), lambda i: (i, 0)),
            pl.BlockSpec((RB, 1), lambda i: (i, 0)),
        ],
        out_shape=[
            jax.ShapeDtypeStruct((N, C), jnp.float32),
            jax.ShapeDtypeStruct((N, 1), jnp.float32),
            jax.ShapeDtypeStruct((N, 1), jnp.float32),
        ],
    )(x, W, deg_partial[0], deg_partial[1])

    p1 = _prop(g0, src, dst, z64, mesh)
    g1 = _combine(p1, g0, d_a)
    p2 = _prop(g1, src, dst, z64, mesh)
    return _combine(p2, g1, d_b, bias=b.reshape(1, C))


# speedup vs baseline: 41.2547x; 1.5352x over previous
import jax
import jax.numpy as jnp
from jax import lax
from jax.experimental import pallas as pl
from jax.experimental.pallas import tpu as pltpu
from jax.experimental.pallas import tpu_sc as plsc

N = 10000
D = 128
E = 320000
C = 64

NC = 2
NS = 16
NW = NC * NS
EPW = E // NW
CHUNK = 80
NCHUNK = EPW // CHUNK
PN = 10240
ROWS_PT = PN // NS
DEG_PAD = 10240
DROWS_PT = DEG_PAD // NS


def _deg_body(dst_hbm, ones_hbm, z_hbm, out_hbm, didx, ones_v, dsem, dacc):
    c = lax.axis_index("c")
    s = lax.axis_index("s")
    wid = c * NS + s
    r0 = s * DROWS_PT
    pltpu.sync_copy(z_hbm.at[pl.ds(r0, DROWS_PT)], dacc.at[pl.ds(r0, DROWS_PT)])
    pltpu.sync_copy(ones_hbm, ones_v)
    pltpu.sync_copy(dst_hbm.at[wid], didx)
    plsc.subcore_barrier()

    def body(j, carry):
        pltpu.async_copy(ones_v, dacc.at[didx.at[j]], dsem, add=True)
        return carry

    lax.fori_loop(0, NCHUNK, body, 0)

    def drain(j, carry):
        pltpu.make_async_copy(ones_v, dacc.at[didx.at[j]], dsem).wait()
        return carry

    lax.fori_loop(0, NCHUNK, drain, 0)
    plsc.subcore_barrier()
    pltpu.sync_copy(dacc.at[pl.ds(r0, DROWS_PT)],
                    out_hbm.at[c, pl.ds(r0, DROWS_PT)])


NBUF = 4


def _prop_body(g_hbm, src_hbm, dst_hbm, z_hbm, out_hbm,
               sidx, didx, bufs, sems, acc):
    c = lax.axis_index("c")
    s = lax.axis_index("s")
    wid = c * NS + s
    r0 = s * ROWS_PT
    pltpu.sync_copy(z_hbm.at[pl.ds(r0, ROWS_PT)], acc.at[pl.ds(r0, ROWS_PT)])
    pltpu.sync_copy(src_hbm.at[wid], sidx)
    pltpu.sync_copy(dst_hbm.at[wid], didx)
    for b in range(NBUF):
        pltpu.async_copy(g_hbm.at[sidx.at[b]], bufs[b], sems[b])
    plsc.subcore_barrier()

    def slot(j, b, refill):
        pltpu.make_async_copy(g_hbm.at[sidx.at[j]], bufs[b], sems[b]).wait()
        pltpu.sync_copy(bufs[b], acc.at[didx.at[j]], add=True)
        if refill:
            pltpu.async_copy(g_hbm.at[sidx.at[j + NBUF]], bufs[b], sems[b])

    def body(i, carry):
        base = NBUF * i
        for b in range(NBUF):
            slot(base + b, b, True)
        return carry

    lax.fori_loop(0, (NCHUNK - NBUF - 1) // NBUF, body, 0)
    slot(NCHUNK - 5, 0, True)
    for j in range(NCHUNK - 4, NCHUNK):
        slot(j, j % NBUF, False)
    plsc.subcore_barrier()
    pltpu.sync_copy(acc.at[pl.ds(r0, ROWS_PT)],
                    out_hbm.at[c, pl.ds(r0, ROWS_PT)])


RB = 1000


def _mm_body(x_ref, w_ref, d0_ref, d1_ref, g0_ref, da_ref, db_ref):
    deg = d0_ref[:, 0:1] + d1_ref[:, 0:1] + 1.0
    dis = lax.rsqrt(deg)
    y = lax.dot_general(x_ref[...], w_ref[...], (((1,), (1,)), ((), ())),
                        preferred_element_type=jnp.float32,
                        precision=lax.Precision.HIGHEST)
    g0_ref[...] = y * dis
    da_ref[...] = 1.0 / deg
    db_ref[...] = dis


def _comb_body(p_ref, g_ref, sc_ref, o_ref):
    p = p_ref[...]
    o_ref[...] = (p[0] + p[1] + g_ref[...]) * sc_ref[...]


def _out_body(p_ref, g_ref, sc_ref, b_ref, o_ref):
    p = p_ref[...]
    o_ref[...] = (p[0] + p[1] + g_ref[...]) * sc_ref[...] + b_ref[...]


def _prop(g, src, dst, z64, mesh):
    return pl.kernel(
        _prop_body,
        out_type=jax.ShapeDtypeStruct((NC, PN, C), jnp.float32),
        mesh=mesh,
        scratch_types=[
            pltpu.VMEM((NCHUNK, CHUNK), jnp.int32),
            pltpu.VMEM((NCHUNK, CHUNK), jnp.int32),
            [pltpu.VMEM((CHUNK, C), jnp.float32) for _ in range(NBUF)],
            [pltpu.SemaphoreType.DMA for _ in range(NBUF)],
            pltpu.VMEM_SHARED((PN, C), jnp.float32),
        ],
        compiler_params=pltpu.CompilerParams(use_tc_tiling_on_sc=False),
    )(g, src, dst, z64)


def _combine(partial, g, scale, bias=None):
    if bias is None:
        body, extra = _comb_body, []
        in_specs_extra = []
    else:
        body, extra = _out_body, [bias]
        in_specs_extra = [pl.BlockSpec((1, C), lambda i: (0, 0))]
    return pl.pallas_call(
        body,
        grid=(N // RB,),
        in_specs=[
            pl.BlockSpec((NC, RB, C), lambda i: (0, i, 0)),
            pl.BlockSpec((RB, C), lambda i: (i, 0)),
            pl.BlockSpec((RB, 1), lambda i: (i, 0)),
        ] + in_specs_extra,
        out_specs=pl.BlockSpec((RB, C), lambda i: (i, 0)),
        out_shape=jax.ShapeDtypeStruct((N, C), jnp.float32),
    )(partial, g, scale, *extra)


def kernel(x, edge_index, W, b):
    ei = edge_index.astype(jnp.int32)
    src = ei[0].reshape(NW, NCHUNK, CHUNK)
    dst = ei[1].reshape(NW, NCHUNK, CHUNK)
    z64 = jnp.zeros((PN, C), jnp.float32)
    z16 = jnp.zeros((DEG_PAD, 16), jnp.float32)
    ones16 = jnp.ones((CHUNK, 16), jnp.float32)
    mesh = plsc.VectorSubcoreMesh(core_axis_name="c", subcore_axis_name="s",
                                  num_cores=NC, num_subcores=NS)

    deg_partial = pl.kernel(
        _deg_body,
        out_type=jax.ShapeDtypeStruct((NC, DEG_PAD, 16), jnp.float32),
        mesh=mesh,
        scratch_types=[
            pltpu.VMEM((NCHUNK, CHUNK), jnp.int32),
            pltpu.VMEM((CHUNK, 16), jnp.float32),
            pltpu.SemaphoreType.DMA,
            pltpu.VMEM_SHARED((DEG_PAD, 16), jnp.float32),
        ],
        compiler_params=pltpu.CompilerParams(use_tc_tiling_on_sc=False),
    )(dst, ones16, z16)

    g0, d_a, d_b = pl.pallas_call(
        _mm_body,
        grid=(N // RB,),
        in_specs=[
            pl.BlockSpec((RB, D), lambda i: (i, 0)),
            pl.BlockSpec((C, D), lambda i: (0, 0)),
            pl.BlockSpec((RB, 16), lambda i: (i, 0)),
            pl.BlockSpec((RB, 16), lambda i: (i, 0)),
        ],
        out_specs=[
            pl.BlockSpec((RB, C), lambda i: (i, 0)),
            pl.BlockSpec((RB, 1), lambda i: (i, 0)),
            pl.BlockSpec((RB, 1), lambda i: (i, 0)),
        ],
        out_shape=[
            jax.ShapeDtypeStruct((N, C), jnp.float32),
            jax.ShapeDtypeStruct((N, 1), jnp.float32),
            jax.ShapeDtypeStruct((N, 1), jnp.float32),
        ],
    )(x, W, deg_partial[0], deg_partial[1])

    p1 = _prop(g0, src, dst, z64, mesh)
    g1 = _combine(p1, g0, d_a)
    p2 = _prop(g1, src, dst, z64, mesh)
    return _combine(p2, g1, d_b, bias=b.reshape(1, C))
